# initial kernel scaffold (unmeasured)
import jax
import jax.numpy as jnp
from jax import lax
from jax.experimental import pallas as pl
from jax.experimental.pallas import tpu as pltpu

N_DEV = 4
E_LOCAL = 8
E_TOTAL = 32
D = 512
H = 1024
T = 2048

_MS = getattr(pltpu, "MemorySpace", None) or pltpu.TPUMemorySpace
_ANY = _MS.ANY
_CompilerParams = getattr(pltpu, "CompilerParams", None) or pltpu.TPUCompilerParams


def kernel(x, router_W, route_idx, expert_W, shared_W):
    def body(x_ref, rw_ref, idx_ref, ew_ref, sw_ref, out_ref,
             comm_ref, wbuf_ref, send_sems, recv_sems, copy_sems):
        me = lax.axis_index("i")

        bar = pltpu.get_barrier_semaphore()
        for r in (1, 2, 3):
            pl.semaphore_signal(
                bar, inc=1,
                device_id=((me + r) % N_DEV,),
                device_id_type=pl.DeviceIdType.MESH,
            )
        pl.semaphore_wait(bar, 3)

        sends = []
        for r in (1, 2, 3):
            s = 3 - r
            rdma = pltpu.make_async_remote_copy(
                src_ref=ew_ref,
                dst_ref=comm_ref.at[s],
                send_sem=send_sems.at[s],
                recv_sem=recv_sems.at[s],
                device_id=((me + r) % N_DEV,),
                device_id_type=pl.DeviceIdType.MESH,
            )
            rdma.start()
            sends.append(rdma)

        xv = x_ref[...]
        scores = jnp.dot(xv, rw_ref[...], preferred_element_type=jnp.float32)
        smax = jnp.max(scores, axis=-1, keepdims=True)
        p = jnp.exp(scores - smax)
        probs = p / jnp.sum(p, axis=-1, keepdims=True)
        idx = idx_ref[...]
        cols = lax.broadcasted_iota(jnp.int32, (T, E_TOTAL), 1)
        gate = jnp.sum(jnp.where(cols == idx, probs, 0.0), axis=1, keepdims=True)

        out_ref[...] = jnp.dot(xv, sw_ref[...], preferred_element_type=jnp.float32)

        def block_contrib(block_ref, origin):
            cp = pltpu.make_async_copy(
                block_ref.at[0], wbuf_ref.at[0], copy_sems.at[0])
            cp.start()
            for j in range(E_LOCAL):
                if j + 1 < E_LOCAL:
                    nxt = pltpu.make_async_copy(
                        block_ref.at[j + 1], wbuf_ref.at[(j + 1) % 2],
                        copy_sems.at[(j + 1) % 2])
                    nxt.start()
                cp.wait()
                y = jnp.dot(xv, wbuf_ref[j % 2],
                            preferred_element_type=jnp.float32)
                e_g = origin * E_LOCAL + j
                coeff = jnp.where(idx == e_g, gate, 0.0)
                out_ref[...] += coeff * y
                if j + 1 < E_LOCAL:
                    cp = nxt

        block_contrib(ew_ref, me)

        for s in range(N_DEV - 1):
            recv = pltpu.make_async_remote_copy(
                src_ref=comm_ref.at[s],
                dst_ref=comm_ref.at[s],
                send_sem=send_sems.at[s],
                recv_sem=recv_sems.at[s],
                device_id=(me,),
                device_id_type=pl.DeviceIdType.MESH,
            )
            recv.wait_recv()
            block_contrib(comm_ref.at[s], (me + s + 1) % N_DEV)

        for rdma in sends:
            rdma.wait_send()

    out_shape = jax.ShapeDtypeStruct((T, H), jnp.float32)
    return pl.pallas_call(
        body,
        out_shape=out_shape,
        in_specs=[
            pl.BlockSpec(memory_space=pltpu.VMEM),
            pl.BlockSpec(memory_space=pltpu.VMEM),
            pl.BlockSpec(memory_space=pltpu.VMEM),
            pl.BlockSpec(memory_space=_ANY),
            pl.BlockSpec(memory_space=pltpu.VMEM),
        ],
        out_specs=pl.BlockSpec(memory_space=pltpu.VMEM),
        scratch_shapes=[
            _ANY((N_DEV - 1, E_LOCAL, D, H), jnp.float32),
            pltpu.VMEM((2, D, H), jnp.float32),
            pltpu.SemaphoreType.DMA((N_DEV - 1,)),
            pltpu.SemaphoreType.DMA((N_DEV - 1,)),
            pltpu.SemaphoreType.DMA((2,)),
        ],
        compiler_params=_CompilerParams(collective_id=0),
    )(x, router_W, route_idx, expert_W, shared_W)


# baseline (device time: 440530 ns/iter reference)
import jax
import jax.numpy as jnp
from jax import lax
from jax.experimental import pallas as pl
from jax.experimental.pallas import tpu as pltpu

N_DEV = 4
E_LOCAL = 8
E_TOTAL = 32
D = 512
H = 1024
T = 2048

_CompilerParams = getattr(pltpu, "CompilerParams", None) or pltpu.TPUCompilerParams


def kernel(x, router_W, route_idx, expert_W, shared_W):
    def body(x_ref, rw_ref, idx_ref, ew_ref, sw_ref, out_ref, comm_ref,
             wbuf_ref, send_sems, recv_sems, copy_sems):
        me = lax.axis_index("i")

        bar = pltpu.get_barrier_semaphore()
        for r in (1, 2, 3):
            pl.semaphore_signal(
                bar, inc=1,
                device_id=((me + r) % N_DEV,),
                device_id_type=pl.DeviceIdType.MESH,
            )
        pl.semaphore_wait(bar, 3)

        sends = []
        for r in (1, 2, 3):
            s = 3 - r
            rdma = pltpu.make_async_remote_copy(
                src_ref=ew_ref,
                dst_ref=comm_ref.at[s],
                send_sem=send_sems.at[s],
                recv_sem=recv_sems.at[s],
                device_id=((me + r) % N_DEV,),
                device_id_type=pl.DeviceIdType.MESH,
            )
            rdma.start()
            sends.append(rdma)

        xv = x_ref[...]
        scores = jnp.dot(xv, rw_ref[...], preferred_element_type=jnp.float32)
        smax = jnp.max(scores, axis=-1, keepdims=True)
        p = jnp.exp(scores - smax)
        probs = p / jnp.sum(p, axis=-1, keepdims=True)
        idx = idx_ref[...]
        cols = lax.broadcasted_iota(jnp.int32, (T, E_TOTAL), 1)
        gate = jnp.sum(jnp.where(cols == idx, probs, 0.0), axis=1, keepdims=True)

        out_ref[...] = jnp.dot(xv, sw_ref[...], preferred_element_type=jnp.float32)

        def block_contrib(block_ref, origin):
            pltpu.make_async_copy(
                block_ref.at[0], wbuf_ref.at[0], copy_sems.at[0]).start()

            def step(j, carry):
                nxt = lax.rem(j + 1, 2)
                cur = lax.rem(j, 2)

                @pl.when(j + 1 < E_LOCAL)
                def _():
                    pltpu.make_async_copy(
                        block_ref.at[j + 1], wbuf_ref.at[nxt],
                        copy_sems.at[nxt]).start()

                pltpu.make_async_copy(
                    block_ref.at[j], wbuf_ref.at[cur], copy_sems.at[cur]).wait()
                y = jnp.dot(xv, wbuf_ref[cur],
                            preferred_element_type=jnp.float32)
                e_g = origin * E_LOCAL + j
                coeff = jnp.where(idx == e_g, gate, 0.0)
                out_ref[...] += coeff * y
                return carry

            lax.fori_loop(0, E_LOCAL, step, 0)

        block_contrib(ew_ref, me)

        for s in range(N_DEV - 1):
            recv = pltpu.make_async_remote_copy(
                src_ref=comm_ref.at[s],
                dst_ref=comm_ref.at[s],
                send_sem=send_sems.at[s],
                recv_sem=recv_sems.at[s],
                device_id=(me,),
                device_id_type=pl.DeviceIdType.MESH,
            )
            recv.wait_recv()
            block_contrib(comm_ref.at[s], (me + s + 1) % N_DEV)

        for rdma in sends:
            rdma.wait_send()

    out_shapes = (
        jax.ShapeDtypeStruct((T, H), jnp.float32),
        jax.ShapeDtypeStruct((N_DEV - 1, E_LOCAL, D, H), jnp.float32),
    )
    out, _comm = pl.pallas_call(
        body,
        out_shape=out_shapes,
        in_specs=[
            pl.BlockSpec(memory_space=pltpu.VMEM),
            pl.BlockSpec(memory_space=pltpu.VMEM),
            pl.BlockSpec(memory_space=pltpu.VMEM),
            pl.BlockSpec(memory_space=pl.ANY),
            pl.BlockSpec(memory_space=pltpu.VMEM),
        ],
        out_specs=(
            pl.BlockSpec(memory_space=pltpu.VMEM),
            pl.BlockSpec(memory_space=pl.ANY),
        ),
        scratch_shapes=[
            pltpu.VMEM((2, D, H), jnp.float32),
            pltpu.SemaphoreType.DMA((N_DEV - 1,)),
            pltpu.SemaphoreType.DMA((N_DEV - 1,)),
            pltpu.SemaphoreType.DMA((2,)),
        ],
        compiler_params=_CompilerParams(collective_id=0),
    )(x, router_W, route_idx, expert_W, shared_W)
    return out


# device time: 266781 ns/iter; 1.6513x vs baseline; 1.6513x over previous
import jax
import jax.numpy as jnp
from jax import lax
from jax.experimental import pallas as pl
from jax.experimental.pallas import tpu as pltpu

N_DEV = 4
E_LOCAL = 8
E_TOTAL = 32
D = 512
H = 1024
T = 2048

_CompilerParams = getattr(pltpu, "CompilerParams", None) or pltpu.TPUCompilerParams


def kernel(x, router_W, route_idx, expert_W, shared_W):
    ew16 = expert_W.astype(jnp.bfloat16)
    sw16 = shared_W.astype(jnp.bfloat16)

    def body(x_ref, rw_ref, idx_ref, ew_ref, sw_ref, out_ref, comm_ref,
             wbuf_ref, send_sems, recv_sems, copy_sems):
        me = lax.axis_index("i")

        bar = pltpu.get_barrier_semaphore()
        for r in (1, 2, 3):
            pl.semaphore_signal(
                bar, inc=1,
                device_id=((me + r) % N_DEV,),
                device_id_type=pl.DeviceIdType.MESH,
            )
        pl.semaphore_wait(bar, 3)

        sends = []
        for r in (1, 2, 3):
            s = 3 - r
            rdma = pltpu.make_async_remote_copy(
                src_ref=ew_ref,
                dst_ref=comm_ref.at[s],
                send_sem=send_sems.at[s],
                recv_sem=recv_sems.at[s],
                device_id=((me + r) % N_DEV,),
                device_id_type=pl.DeviceIdType.MESH,
            )
            rdma.start()
            sends.append(rdma)

        xv = x_ref[...]
        xv16 = xv.astype(jnp.bfloat16)
        scores = jnp.dot(xv, rw_ref[...], preferred_element_type=jnp.float32)
        smax = jnp.max(scores, axis=-1, keepdims=True)
        p = jnp.exp(scores - smax)
        probs = p / jnp.sum(p, axis=-1, keepdims=True)
        idx = idx_ref[...]
        cols = lax.broadcasted_iota(jnp.int32, (T, E_TOTAL), 1)
        gate = jnp.sum(jnp.where(cols == idx, probs, 0.0), axis=1, keepdims=True)

        out_ref[...] = jnp.dot(xv16, sw_ref[...],
                               preferred_element_type=jnp.float32)

        def block_contrib(block_ref, origin):
            pltpu.make_async_copy(
                block_ref.at[0], wbuf_ref.at[0], copy_sems.at[0]).start()

            def step(j, carry):
                nxt = lax.rem(j + 1, 2)
                cur = lax.rem(j, 2)

                @pl.when(j + 1 < E_LOCAL)
                def _():
                    pltpu.make_async_copy(
                        block_ref.at[j + 1], wbuf_ref.at[nxt],
                        copy_sems.at[nxt]).start()

                pltpu.make_async_copy(
                    block_ref.at[j], wbuf_ref.at[cur], copy_sems.at[cur]).wait()
                y = jnp.dot(xv16, wbuf_ref[cur],
                            preferred_element_type=jnp.float32)
                e_g = origin * E_LOCAL + j
                coeff = jnp.where(idx == e_g, gate, 0.0)
                out_ref[...] += coeff * y
                return carry

            lax.fori_loop(0, E_LOCAL, step, 0)

        block_contrib(ew_ref, me)

        for s in range(N_DEV - 1):
            recv = pltpu.make_async_remote_copy(
                src_ref=comm_ref.at[s],
                dst_ref=comm_ref.at[s],
                send_sem=send_sems.at[s],
                recv_sem=recv_sems.at[s],
                device_id=(me,),
                device_id_type=pl.DeviceIdType.MESH,
            )
            recv.wait_recv()
            block_contrib(comm_ref.at[s], (me + s + 1) % N_DEV)

        for rdma in sends:
            rdma.wait_send()

    out_shapes = (
        jax.ShapeDtypeStruct((T, H), jnp.float32),
        jax.ShapeDtypeStruct((N_DEV - 1, E_LOCAL, D, H), jnp.bfloat16),
    )
    out, _comm = pl.pallas_call(
        body,
        out_shape=out_shapes,
        in_specs=[
            pl.BlockSpec(memory_space=pltpu.VMEM),
            pl.BlockSpec(memory_space=pltpu.VMEM),
            pl.BlockSpec(memory_space=pltpu.VMEM),
            pl.BlockSpec(memory_space=pl.ANY),
            pl.BlockSpec(memory_space=pltpu.VMEM),
        ],
        out_specs=(
            pl.BlockSpec(memory_space=pltpu.VMEM),
            pl.BlockSpec(memory_space=pl.ANY),
        ),
        scratch_shapes=[
            pltpu.VMEM((2, D, H), jnp.bfloat16),
            pltpu.SemaphoreType.DMA((N_DEV - 1,)),
            pltpu.SemaphoreType.DMA((N_DEV - 1,)),
            pltpu.SemaphoreType.DMA((2,)),
        ],
        compiler_params=_CompilerParams(collective_id=0),
    )(x, router_W, route_idx, ew16, sw16)
    return out


# device time: 229014 ns/iter; 1.9236x vs baseline; 1.1649x over previous
import jax
import jax.numpy as jnp
from jax import lax
from jax.experimental import pallas as pl
from jax.experimental.pallas import tpu as pltpu

N_DEV = 4
E_LOCAL = 8
E_TOTAL = 32
D = 512
H = 1024
T = 2048
N_CHUNK = 4
E_CHUNK = E_LOCAL // N_CHUNK

_CompilerParams = getattr(pltpu, "CompilerParams", None) or pltpu.TPUCompilerParams


def kernel(x, router_W, route_idx, expert_W, shared_W):
    ew16 = expert_W.astype(jnp.bfloat16)
    sw16 = shared_W.astype(jnp.bfloat16)

    def body(x_ref, rw_ref, idx_ref, ew_ref, sw_ref, out_ref,
             comm_ref, wbuf_ref, send_sems, recv_sems, copy_sems):
        me = lax.axis_index("i")

        bar = pltpu.get_barrier_semaphore()
        for r in (1, 2, 3):
            pl.semaphore_signal(
                bar, inc=1,
                device_id=((me + r) % N_DEV,),
                device_id_type=pl.DeviceIdType.MESH,
            )
        pl.semaphore_wait(bar, 3)

        sends = []
        for c in range(N_CHUNK):
            lo = c * E_CHUNK
            for r in (1, 2, 3):
                s = 3 - r
                rdma = pltpu.make_async_remote_copy(
                    src_ref=ew_ref.at[pl.ds(lo, E_CHUNK)],
                    dst_ref=comm_ref.at[s, pl.ds(lo, E_CHUNK)],
                    send_sem=send_sems.at[s, c],
                    recv_sem=recv_sems.at[s, c],
                    device_id=((me + r) % N_DEV,),
                    device_id_type=pl.DeviceIdType.MESH,
                )
                rdma.start()
                sends.append(rdma)

        xv = x_ref[...]
        xv16 = xv.astype(jnp.bfloat16)
        scores = jnp.dot(xv, rw_ref[...], preferred_element_type=jnp.float32)
        smax = jnp.max(scores, axis=-1, keepdims=True)
        p = jnp.exp(scores - smax)
        probs = p / jnp.sum(p, axis=-1, keepdims=True)
        idx = idx_ref[...]
        cols = lax.broadcasted_iota(jnp.int32, (T, E_TOTAL), 1)
        gate = jnp.sum(jnp.where(cols == idx, probs, 0.0), axis=1, keepdims=True)

        out_ref[...] = jnp.dot(xv16, sw_ref[...],
                               preferred_element_type=jnp.float32)

        def acc_expert(w16, e_g):
            y = jnp.dot(xv16, w16, preferred_element_type=jnp.float32)
            coeff = jnp.where(idx == e_g, gate, 0.0)
            out_ref[...] += coeff * y

        pltpu.make_async_copy(
            ew_ref.at[0], wbuf_ref.at[0], copy_sems.at[0]).start()

        def local_step(j, carry):
            nxt = lax.rem(j + 1, 2)
            cur = lax.rem(j, 2)

            @pl.when(j + 1 < E_LOCAL)
            def _():
                pltpu.make_async_copy(
                    ew_ref.at[j + 1], wbuf_ref.at[nxt],
                    copy_sems.at[nxt]).start()

            pltpu.make_async_copy(
                ew_ref.at[j], wbuf_ref.at[cur], copy_sems.at[cur]).wait()
            acc_expert(wbuf_ref[cur], me * E_LOCAL + j)
            return carry

        lax.fori_loop(0, E_LOCAL, local_step, 0)

        def remote_step(k, carry):
            s = lax.rem(k, N_DEV - 1)
            c = k // (N_DEV - 1)
            lo = c * E_CHUNK
            recv = pltpu.make_async_remote_copy(
                src_ref=comm_ref.at[s, pl.ds(lo, E_CHUNK)],
                dst_ref=comm_ref.at[s, pl.ds(lo, E_CHUNK)],
                send_sem=send_sems.at[s, c],
                recv_sem=recv_sems.at[s, c],
                device_id=(me,),
                device_id_type=pl.DeviceIdType.MESH,
            )
            recv.wait_recv()
            origin = lax.rem(me + s + 1, N_DEV)
            for jj in range(E_CHUNK):
                acc_expert(comm_ref[s, lo + jj], origin * E_LOCAL + lo + jj)
            return carry

        lax.fori_loop(0, (N_DEV - 1) * N_CHUNK, remote_step, 0)

        for rdma in sends:
            rdma.wait_send()

    out_shape = jax.ShapeDtypeStruct((T, H), jnp.float32)
    out = pl.pallas_call(
        body,
        out_shape=out_shape,
        in_specs=[
            pl.BlockSpec(memory_space=pltpu.VMEM),
            pl.BlockSpec(memory_space=pltpu.VMEM),
            pl.BlockSpec(memory_space=pltpu.VMEM),
            pl.BlockSpec(memory_space=pl.ANY),
            pl.BlockSpec(memory_space=pltpu.VMEM),
        ],
        out_specs=pl.BlockSpec(memory_space=pltpu.VMEM),
        scratch_shapes=[
            pltpu.VMEM((N_DEV - 1, E_LOCAL, D, H), jnp.bfloat16),
            pltpu.VMEM((2, D, H), jnp.bfloat16),
            pltpu.SemaphoreType.DMA((N_DEV - 1, N_CHUNK)),
            pltpu.SemaphoreType.DMA((N_DEV - 1, N_CHUNK)),
            pltpu.SemaphoreType.DMA((2,)),
        ],
        compiler_params=_CompilerParams(
            collective_id=0, vmem_limit_bytes=60 * 1024 * 1024),
    )(x, router_W, route_idx, ew16, sw16)
    return out


# device time: 190375 ns/iter; 2.3140x vs baseline; 1.2030x over previous
import jax
import jax.numpy as jnp
from jax import lax
from jax.experimental import pallas as pl
from jax.experimental.pallas import tpu as pltpu

N_DEV = 4
E_LOCAL = 8
E_HALF = E_LOCAL // 2
E_TOTAL = 32
D = 512
H = 1024
T = 2048
N_HOP = N_DEV - 1

_CompilerParams = getattr(pltpu, "CompilerParams", None) or pltpu.TPUCompilerParams


def kernel(x, router_W, route_idx, expert_W, shared_W):
    ew16 = expert_W.astype(jnp.bfloat16)
    sw16 = shared_W.astype(jnp.bfloat16)

    def body(x_ref, rw_ref, idx_ref, ew_ref, sw_ref, out_ref,
             cw_ref, ccw_ref, wbuf_ref,
             cw_send, cw_recv, ccw_send, ccw_recv, copy_sems):
        me = lax.axis_index("i")
        right = lax.rem(me + 1, N_DEV)
        left = lax.rem(me + N_DEV - 1, N_DEV)

        bar = pltpu.get_barrier_semaphore()
        for r in (1, 2, 3):
            pl.semaphore_signal(
                bar, inc=1,
                device_id=(lax.rem(me + r, N_DEV),),
                device_id_type=pl.DeviceIdType.MESH,
            )
        pl.semaphore_wait(bar, 3)

        def send(src_ref, dst_ref, ssem, rsem, dev):
            rdma = pltpu.make_async_remote_copy(
                src_ref=src_ref, dst_ref=dst_ref, send_sem=ssem,
                recv_sem=rsem, device_id=(dev,),
                device_id_type=pl.DeviceIdType.MESH)
            rdma.start()
            return rdma

        def wait_recv(buf_ref, rsem, dummy_ssem):
            pltpu.make_async_remote_copy(
                src_ref=buf_ref, dst_ref=buf_ref, send_sem=dummy_ssem,
                recv_sem=rsem, device_id=(me,),
                device_id_type=pl.DeviceIdType.MESH).wait_recv()

        sends = [
            send(ew_ref.at[pl.ds(0, E_HALF)], cw_ref.at[0],
                 cw_send.at[0], cw_recv.at[0], right),
            send(ew_ref.at[pl.ds(E_HALF, E_HALF)], ccw_ref.at[0],
                 ccw_send.at[0], ccw_recv.at[0], left),
        ]

        xv = x_ref[...]
        xv16 = xv.astype(jnp.bfloat16)
        scores = jnp.dot(xv, rw_ref[...], preferred_element_type=jnp.float32)
        smax = jnp.max(scores, axis=-1, keepdims=True)
        p = jnp.exp(scores - smax)
        probs = p / jnp.sum(p, axis=-1, keepdims=True)
        idx = idx_ref[...]
        cols = lax.broadcasted_iota(jnp.int32, (T, E_TOTAL), 1)
        gate = jnp.sum(jnp.where(cols == idx, probs, 0.0), axis=1, keepdims=True)

        out_ref[...] = jnp.dot(xv16, sw_ref[...],
                               preferred_element_type=jnp.float32)

        def acc_expert(w16, e_g):
            y = jnp.dot(xv16, w16, preferred_element_type=jnp.float32)
            coeff = jnp.where(idx == e_g, gate, 0.0)
            out_ref[...] += coeff * y

        pltpu.make_async_copy(
            ew_ref.at[0], wbuf_ref.at[0], copy_sems.at[0]).start()

        def local_step(j, carry):
            nxt = lax.rem(j + 1, 2)
            cur = lax.rem(j, 2)

            @pl.when(j + 1 < E_LOCAL)
            def _():
                pltpu.make_async_copy(
                    ew_ref.at[j + 1], wbuf_ref.at[nxt],
                    copy_sems.at[nxt]).start()

            pltpu.make_async_copy(
                ew_ref.at[j], wbuf_ref.at[cur], copy_sems.at[cur]).wait()
            acc_expert(wbuf_ref[cur], me * E_LOCAL + j)
            return carry

        lax.fori_loop(0, E_LOCAL, local_step, 0)

        def hop_step(k, carry):
            kn = jnp.minimum(k + 1, N_HOP - 1)
            wait_recv(cw_ref.at[k], cw_recv.at[k], cw_send.at[k])

            @pl.when(k < N_HOP - 1)
            def _():
                send(cw_ref.at[k], cw_ref.at[kn],
                     cw_send.at[kn], cw_recv.at[kn], right)

            wait_recv(ccw_ref.at[k], ccw_recv.at[k], ccw_send.at[k])

            @pl.when(k < N_HOP - 1)
            def _():
                send(ccw_ref.at[k], ccw_ref.at[kn],
                     ccw_send.at[kn], ccw_recv.at[kn], left)

            origin_cw = lax.rem(me + N_DEV - k - 1, N_DEV)
            origin_ccw = lax.rem(me + k + 1, N_DEV)
            for jj in range(E_HALF):
                acc_expert(cw_ref[k, jj], origin_cw * E_LOCAL + jj)
                acc_expert(ccw_ref[k, jj], origin_ccw * E_LOCAL + E_HALF + jj)
            return carry

        lax.fori_loop(0, N_HOP, hop_step, 0)

        for rdma in sends:
            rdma.wait_send()
        for k in range(1, N_HOP):
            pltpu.make_async_remote_copy(
                src_ref=cw_ref.at[k - 1], dst_ref=cw_ref.at[k],
                send_sem=cw_send.at[k], recv_sem=cw_recv.at[k],
                device_id=(right,),
                device_id_type=pl.DeviceIdType.MESH).wait_send()
            pltpu.make_async_remote_copy(
                src_ref=ccw_ref.at[k - 1], dst_ref=ccw_ref.at[k],
                send_sem=ccw_send.at[k], recv_sem=ccw_recv.at[k],
                device_id=(left,),
                device_id_type=pl.DeviceIdType.MESH).wait_send()

    out_shape = jax.ShapeDtypeStruct((T, H), jnp.float32)
    out = pl.pallas_call(
        body,
        out_shape=out_shape,
        in_specs=[
            pl.BlockSpec(memory_space=pltpu.VMEM),
            pl.BlockSpec(memory_space=pltpu.VMEM),
            pl.BlockSpec(memory_space=pltpu.VMEM),
            pl.BlockSpec(memory_space=pl.ANY),
            pl.BlockSpec(memory_space=pltpu.VMEM),
        ],
        out_specs=pl.BlockSpec(memory_space=pltpu.VMEM),
        scratch_shapes=[
            pltpu.VMEM((N_HOP, E_HALF, D, H), jnp.bfloat16),
            pltpu.VMEM((N_HOP, E_HALF, D, H), jnp.bfloat16),
            pltpu.VMEM((2, D, H), jnp.bfloat16),
            pltpu.SemaphoreType.DMA((N_HOP,)),
            pltpu.SemaphoreType.DMA((N_HOP,)),
            pltpu.SemaphoreType.DMA((N_HOP,)),
            pltpu.SemaphoreType.DMA((N_HOP,)),
            pltpu.SemaphoreType.DMA((2,)),
        ],
        compiler_params=_CompilerParams(
            collective_id=0, vmem_limit_bytes=60 * 1024 * 1024),
    )(x, router_W, route_idx, ew16, sw16)
    return out


# device time: 179166 ns/iter; 2.4588x vs baseline; 1.0626x over previous
import jax
import jax.numpy as jnp
from jax import lax
from jax.experimental import pallas as pl
from jax.experimental.pallas import tpu as pltpu

N_DEV = 4
E_LOCAL = 8
E_HALF = E_LOCAL // 2
E_CHUNK = 2
C_HALF = E_HALF // E_CHUNK
E_TOTAL = 32
D = 512
H = 1024
T = 2048
N_HOP = N_DEV - 1

_CompilerParams = getattr(pltpu, "CompilerParams", None) or pltpu.TPUCompilerParams


def kernel(x, router_W, route_idx, expert_W, shared_W):
    ew16 = expert_W.astype(jnp.bfloat16)
    sw16 = shared_W.astype(jnp.bfloat16)

    def body(x_ref, rw_ref, idx_ref, ew_ref, sw_ref, out_ref,
             cw_ref, ccw_ref, wbuf_ref,
             cw_send, cw_recv, ccw_send, ccw_recv, copy_sems):
        me = lax.axis_index("i")
        right = lax.rem(me + 1, N_DEV)
        left = lax.rem(me + N_DEV - 1, N_DEV)

        bar = pltpu.get_barrier_semaphore()
        for r in (1, 2, 3):
            pl.semaphore_signal(
                bar, inc=1,
                device_id=(lax.rem(me + r, N_DEV),),
                device_id_type=pl.DeviceIdType.MESH,
            )
        pl.semaphore_wait(bar, 3)

        def send(src_ref, dst_ref, ssem, rsem, dev):
            rdma = pltpu.make_async_remote_copy(
                src_ref=src_ref, dst_ref=dst_ref, send_sem=ssem,
                recv_sem=rsem, device_id=(dev,),
                device_id_type=pl.DeviceIdType.MESH)
            rdma.start()
            return rdma

        def wait_recv(buf_ref, rsem, dummy_ssem):
            pltpu.make_async_remote_copy(
                src_ref=buf_ref, dst_ref=buf_ref, send_sem=dummy_ssem,
                recv_sem=rsem, device_id=(me,),
                device_id_type=pl.DeviceIdType.MESH).wait_recv()

        sends = []
        for c in range(C_HALF):
            sends.append(send(
                ew_ref.at[pl.ds(c * E_CHUNK, E_CHUNK)],
                cw_ref.at[0, pl.ds(c * E_CHUNK, E_CHUNK)],
                cw_send.at[0, c], cw_recv.at[0, c], right))
            sends.append(send(
                ew_ref.at[pl.ds(E_HALF + c * E_CHUNK, E_CHUNK)],
                ccw_ref.at[0, pl.ds(c * E_CHUNK, E_CHUNK)],
                ccw_send.at[0, c], ccw_recv.at[0, c], left))

        xv = x_ref[...]
        xv16 = xv.astype(jnp.bfloat16)
        scores = jnp.dot(xv, rw_ref[...], preferred_element_type=jnp.float32)
        smax = jnp.max(scores, axis=-1, keepdims=True)
        p = jnp.exp(scores - smax)
        probs = p / jnp.sum(p, axis=-1, keepdims=True)
        idx = idx_ref[...]
        cols = lax.broadcasted_iota(jnp.int32, (T, E_TOTAL), 1)
        gate = jnp.sum(jnp.where(cols == idx, probs, 0.0), axis=1, keepdims=True)
        gate16 = gate.astype(jnp.bfloat16)

        out_ref[...] = jnp.dot(xv16, sw_ref[...],
                               preferred_element_type=jnp.float32)

        def acc_chunk(w2_ref, e0):
            parts = []
            for jj in range(E_CHUNK):
                coeff = jnp.where(idx == e0 + jj, gate, 0.0)
                parts.append((xv * coeff).astype(jnp.bfloat16))
            xcat = jnp.concatenate(parts, axis=1)
            w2 = w2_ref[...].reshape(E_CHUNK * D, H)
            out_ref[...] += jnp.dot(xcat, w2,
                                    preferred_element_type=jnp.float32)

        def local_step(j, carry):
            cp = pltpu.make_async_copy(
                ew_ref.at[pl.ds(j * E_CHUNK, E_CHUNK)], wbuf_ref,
                copy_sems.at[0])
            cp.start()
            cp.wait()
            acc_chunk(wbuf_ref, me * E_LOCAL + j * E_CHUNK)
            return carry

        lax.fori_loop(0, E_LOCAL // E_CHUNK, local_step, 0)

        def ring_step(s, carry):
            k = s // C_HALF
            c = lax.rem(s, C_HALF)
            kn = jnp.minimum(k + 1, N_HOP - 1)
            lo = c * E_CHUNK

            wait_recv(cw_ref.at[k, pl.ds(lo, E_CHUNK)],
                      cw_recv.at[k, c], cw_send.at[k, c])

            @pl.when(k < N_HOP - 1)
            def _():
                send(cw_ref.at[k, pl.ds(lo, E_CHUNK)],
                     cw_ref.at[kn, pl.ds(lo, E_CHUNK)],
                     cw_send.at[kn, c], cw_recv.at[kn, c], right)

            wait_recv(ccw_ref.at[k, pl.ds(lo, E_CHUNK)],
                      ccw_recv.at[k, c], ccw_send.at[k, c])

            @pl.when(k < N_HOP - 1)
            def _():
                send(ccw_ref.at[k, pl.ds(lo, E_CHUNK)],
                     ccw_ref.at[kn, pl.ds(lo, E_CHUNK)],
                     ccw_send.at[kn, c], ccw_recv.at[kn, c], left)

            origin_cw = lax.rem(me + N_DEV - k - 1, N_DEV)
            origin_ccw = lax.rem(me + k + 1, N_DEV)
            acc_chunk(cw_ref.at[k, pl.ds(lo, E_CHUNK)],
                      origin_cw * E_LOCAL + lo)
            acc_chunk(ccw_ref.at[k, pl.ds(lo, E_CHUNK)],
                      origin_ccw * E_LOCAL + E_HALF + lo)
            return carry

        lax.fori_loop(0, N_HOP * C_HALF, ring_step, 0)

        for rdma in sends:
            rdma.wait_send()
        for k in range(1, N_HOP):
            for c in range(C_HALF):
                lo = c * E_CHUNK
                pltpu.make_async_remote_copy(
                    src_ref=cw_ref.at[k - 1, pl.ds(lo, E_CHUNK)],
                    dst_ref=cw_ref.at[k, pl.ds(lo, E_CHUNK)],
                    send_sem=cw_send.at[k, c], recv_sem=cw_recv.at[k, c],
                    device_id=(right,),
                    device_id_type=pl.DeviceIdType.MESH).wait_send()
                pltpu.make_async_remote_copy(
                    src_ref=ccw_ref.at[k - 1, pl.ds(lo, E_CHUNK)],
                    dst_ref=ccw_ref.at[k, pl.ds(lo, E_CHUNK)],
                    send_sem=ccw_send.at[k, c], recv_sem=ccw_recv.at[k, c],
                    device_id=(left,),
                    device_id_type=pl.DeviceIdType.MESH).wait_send()

    out_shape = jax.ShapeDtypeStruct((T, H), jnp.float32)
    out = pl.pallas_call(
        body,
        out_shape=out_shape,
        in_specs=[
            pl.BlockSpec(memory_space=pltpu.VMEM),
            pl.BlockSpec(memory_space=pltpu.VMEM),
            pl.BlockSpec(memory_space=pltpu.VMEM),
            pl.BlockSpec(memory_space=pl.ANY),
            pl.BlockSpec(memory_space=pltpu.VMEM),
        ],
        out_specs=pl.BlockSpec(memory_space=pltpu.VMEM),
        scratch_shapes=[
            pltpu.VMEM((N_HOP, E_HALF, D, H), jnp.bfloat16),
            pltpu.VMEM((N_HOP, E_HALF, D, H), jnp.bfloat16),
            pltpu.VMEM((E_CHUNK, D, H), jnp.bfloat16),
            pltpu.SemaphoreType.DMA((N_HOP, C_HALF)),
            pltpu.SemaphoreType.DMA((N_HOP, C_HALF)),
            pltpu.SemaphoreType.DMA((N_HOP, C_HALF)),
            pltpu.SemaphoreType.DMA((N_HOP, C_HALF)),
            pltpu.SemaphoreType.DMA((1,)),
        ],
        compiler_params=_CompilerParams(
            collective_id=0, vmem_limit_bytes=int(63.5 * 1024 * 1024)),
    )(x, router_W, route_idx, ew16, sw16)
    return out
